# baseline (speedup 1.0000x reference)
import functools

import jax
import jax.numpy as jnp
from jax import lax
from jax.experimental import pallas as pl
from jax.experimental.pallas import tpu as pltpu
from jax.experimental.pallas import tpu_sc as plsc

NC = 2
NS = 16
NW = NC * NS
LANES = 16
NBUF = 7


def _make_pool(B, H, V, D):
    rows = B // NW
    mesh = plsc.VectorSubcoreMesh(core_axis_name="c", subcore_axis_name="s")

    @functools.partial(
        pl.kernel,
        out_type=jax.ShapeDtypeStruct((B, D), jnp.float32),
        mesh=mesh,
        compiler_params=pltpu.CompilerParams(
            use_tc_tiling_on_sc=False, needs_layout_passes=False
        ),
        scratch_types=[
            pltpu.VMEM((rows, H), jnp.int32),
            pltpu.VMEM((H, rows), jnp.int32),
            pltpu.VMEM((NBUF, rows, D), jnp.float32),
            pltpu.VMEM((rows,), jnp.int32),
            pltpu.SemaphoreType.DMA((NBUF,)),
            pltpu.VMEM_SHARED((NS * rows, D), jnp.float32),
        ],
    )
    def pool(x_hbm, table_hbm, out_hbm, idx_raw, idx_v, bufs, dst_idx, gsem, acc):
        c = lax.axis_index("c")
        s = lax.axis_index("s")
        wid = c * NS + s
        gbase = wid * rows
        lbase = s * rows

        pltpu.sync_copy(x_hbm.at[pl.ds(gbase, rows)], idx_raw)
        iota = lax.iota(jnp.int32, LANES)

        @pl.loop(0, H)
        def _(j):
            col = jnp.zeros((LANES,), jnp.int32) + j
            for k in range(rows // LANES):
                v = plsc.load_gather(idx_raw, [k * LANES + iota, col])
                idx_v[j, pl.ds(k * LANES, LANES)] = v

        for k in range(rows // LANES):
            dst_idx[pl.ds(k * LANES, LANES)] = (
                lbase + k * LANES + lax.iota(jnp.int32, LANES)
            )

        pltpu.async_copy(table_hbm.at[idx_v.at[0]], bufs.at[0], gsem.at[0]).wait()
        pltpu.sync_copy(bufs.at[0], acc.at[pl.ds(lbase, rows)])

        for b in range(NBUF):
            pltpu.async_copy(table_hbm.at[idx_v.at[1 + b]], bufs.at[b], gsem.at[b])

        n_main = (H - 1 - NBUF) // NBUF

        @pl.loop(0, n_main)
        def _(g):
            for b in range(NBUF):
                j = 1 + g * NBUF + b
                pltpu.make_async_copy(
                    table_hbm.at[idx_v.at[j]], bufs.at[b], gsem.at[b]
                ).wait()
                pltpu.sync_copy(bufs.at[b], acc.at[dst_idx], add=True)
                pltpu.async_copy(
                    table_hbm.at[idx_v.at[j + NBUF]], bufs.at[b], gsem.at[b]
                )

        for b in range(NBUF):
            j = 1 + n_main * NBUF + b
            pltpu.make_async_copy(
                table_hbm.at[idx_v.at[j]], bufs.at[b], gsem.at[b]
            ).wait()
            pltpu.sync_copy(bufs.at[b], acc.at[dst_idx], add=True)

        pltpu.sync_copy(acc.at[pl.ds(lbase, rows)], bufs.at[0])
        pltpu.sync_copy(bufs.at[0], out_hbm.at[pl.ds(gbase, rows)])

    return pool


def _mlp_body(scale, n_cls, pool_ref, w1_ref, b1_ref, w2_ref, b2_ref, out_ref):
    p = pool_ref[...] * scale
    h = lax.dot_general(
        p, w1_ref[...], (((1,), (1,)), ((), ())),
        preferred_element_type=jnp.float32,
    ) + b1_ref[...]
    h = jnp.maximum(h, 0.0)
    o = lax.dot_general(
        h, w2_ref[...], (((1,), (1,)), ((), ())),
        preferred_element_type=jnp.float32,
    ) + b2_ref[...]
    out_ref[...] = o[:, :n_cls]


def kernel(x, emb_table, W1, b1, W2, b2):
    B, H = x.shape
    V, D = emb_table.shape
    HID = W1.shape[0]
    C = W2.shape[0]
    CP = ((C + 127) // 128) * 128

    pooled_sum = _make_pool(B, H, V, D)(jnp.asarray(x, jnp.int32), emb_table)

    W2p = jnp.pad(W2, ((0, CP - C), (0, 0)))
    b2p = jnp.pad(b2, (0, CP - C)).reshape(1, CP)
    b1r = b1.reshape(1, HID)

    BB = 512
    out = pl.pallas_call(
        functools.partial(_mlp_body, 1.0 / H, C),
        grid=(B // BB,),
        in_specs=[
            pl.BlockSpec((BB, D), lambda i: (i, 0)),
            pl.BlockSpec((HID, D), lambda i: (0, 0)),
            pl.BlockSpec((1, HID), lambda i: (0, 0)),
            pl.BlockSpec((CP, HID), lambda i: (0, 0)),
            pl.BlockSpec((1, CP), lambda i: (0, 0)),
        ],
        out_specs=pl.BlockSpec((BB, C), lambda i: (i, 0)),
        out_shape=jax.ShapeDtypeStruct((B, C), jnp.float32),
    )(pooled_sum, W1, b1r, W2p, b2p)

    return out


# speedup vs baseline: 1.5149x; 1.5149x over previous
import functools

import jax
import jax.numpy as jnp
from jax import lax
from jax.experimental import pallas as pl
from jax.experimental.pallas import tpu as pltpu
from jax.experimental.pallas import tpu_sc as plsc

NC = 2
NS = 16
NW = NC * NS
LANES = 16
NBUF = 8
TRB = 8192


def _make_pool(B, H, V, D):
    rows = B // NW
    mesh = plsc.VectorSubcoreMesh(core_axis_name="c", subcore_axis_name="s")

    @functools.partial(
        pl.kernel,
        out_type=jax.ShapeDtypeStruct((B, D), jnp.float32),
        mesh=mesh,
        compiler_params=pltpu.CompilerParams(
            use_tc_tiling_on_sc=False, needs_layout_passes=False
        ),
        scratch_types=[
            pltpu.VMEM((H, rows), jnp.int32),
            pltpu.VMEM((NBUF, rows, D), jnp.float32),
            pltpu.VMEM((rows,), jnp.int32),
            pltpu.SemaphoreType.DMA((NBUF,)),
            pltpu.SemaphoreType.DMA((NBUF,)),
            pltpu.VMEM_SHARED((NS * rows, D), jnp.float32),
        ],
    )
    def pool(xT_hbm, table_hbm, out_hbm, idx_v, bufs, dst_idx, gsem, ssem, acc):
        c = lax.axis_index("c")
        s = lax.axis_index("s")
        wid = c * NS + s
        gbase = wid * rows
        lbase = s * rows

        pltpu.sync_copy(xT_hbm.at[:, pl.ds(gbase, rows)], idx_v)

        @pl.loop(0, H)
        def _(j):
            for k in range(rows // LANES):
                v = idx_v[j, pl.ds(k * LANES, LANES)]
                cc = v & (TRB - 1)
                idx_v[j, pl.ds(k * LANES, LANES)] = (
                    (v - cc) + 2 * cc - jnp.where(cc < TRB // 2, 0, TRB - 1)
                )

        for k in range(rows // LANES):
            dst_idx[pl.ds(k * LANES, LANES)] = (
                lbase + k * LANES + lax.iota(jnp.int32, LANES)
            )

        for b in range(NBUF - 1):
            pltpu.async_copy(table_hbm.at[idx_v.at[1 + b]], bufs.at[b], gsem.at[b])

        pltpu.async_copy(
            table_hbm.at[idx_v.at[0]], bufs.at[NBUF - 1], gsem.at[NBUF - 1]
        ).wait()
        pltpu.sync_copy(bufs.at[NBUF - 1], acc.at[pl.ds(lbase, rows)])

        @pl.loop(1, H)
        def _(j):
            b = (j - 1) % NBUF
            pltpu.make_async_copy(
                table_hbm.at[idx_v.at[j]], bufs.at[b], gsem.at[b]
            ).wait()
            pltpu.async_copy(bufs.at[b], acc.at[dst_idx], ssem.at[b], add=True)
            bp = (j + NBUF - 2) % NBUF
            p = j + NBUF - 1

            @pl.when(jnp.logical_and(j >= 2, p <= H - 1))
            def _():
                pltpu.make_async_copy(
                    bufs.at[bp], acc.at[dst_idx], ssem.at[bp]
                ).wait()

            @pl.when(p <= H - 1)
            def _():
                pltpu.async_copy(
                    table_hbm.at[idx_v.at[p]], bufs.at[bp], gsem.at[bp]
                )

        for b in range(NBUF):
            pltpu.make_async_copy(bufs.at[b], acc.at[dst_idx], ssem.at[b]).wait()

        pltpu.sync_copy(acc.at[pl.ds(lbase, rows)], bufs.at[0])
        pltpu.sync_copy(bufs.at[0], out_hbm.at[pl.ds(gbase, rows)])

    return pool


def _tr_body(tin_ref, out_ref):
    v = tin_ref[...]
    t = jnp.swapaxes(v, 0, 1)
    bc = t.shape[0]
    out_ref[...] = jnp.concatenate([t[: bc // 2], t[bc // 2 :]], axis=1)


def _transpose_to_linear(tableT, V, D):
    BC = TRB
    grid = ((V + BC - 1) // BC,)
    vp = grid[0] * BC
    out = pl.pallas_call(
        _tr_body,
        grid=grid,
        in_specs=[pl.BlockSpec((D, BC), lambda i: (0, i))],
        out_specs=pl.BlockSpec((BC // 2, 2 * D), lambda i: (i, 0)),
        out_shape=jax.ShapeDtypeStruct((vp // 2, 2 * D), jnp.float32),
    )(tableT)
    return out, vp


def _mlp_body(scale, n_cls, pool_ref, w1_ref, b1_ref, w2_ref, b2_ref, out_ref):
    p = pool_ref[...] * scale
    h = lax.dot_general(
        p, w1_ref[...], (((1,), (1,)), ((), ())),
        preferred_element_type=jnp.float32,
    ) + b1_ref[...]
    h = jnp.maximum(h, 0.0)
    o = lax.dot_general(
        h, w2_ref[...], (((1,), (1,)), ((), ())),
        preferred_element_type=jnp.float32,
    ) + b2_ref[...]
    out_ref[...] = jnp.swapaxes(o, 0, 1)[:n_cls, :]


def kernel(x, emb_table, W1, b1, W2, b2):
    B, H = x.shape
    V, D = emb_table.shape
    HID = W1.shape[0]
    C = W2.shape[0]
    CP = ((C + 127) // 128) * 128

    emb_lin, vp = _transpose_to_linear(emb_table.T, V, D)
    pooled_sum = _make_pool(B, H, vp, D)(
        jnp.asarray(x, jnp.int32).T, emb_lin.reshape(vp, D)
    )

    W2p = jnp.pad(W2, ((0, CP - C), (0, 0)))
    b2p = jnp.pad(b2, (0, CP - C)).reshape(1, CP)
    b1r = b1.reshape(1, HID)

    BB = 2048
    out = pl.pallas_call(
        functools.partial(_mlp_body, 1.0 / H, C),
        grid=(B // BB,),
        in_specs=[
            pl.BlockSpec((BB, D), lambda i: (i, 0)),
            pl.BlockSpec((HID, D), lambda i: (0, 0)),
            pl.BlockSpec((1, HID), lambda i: (0, 0)),
            pl.BlockSpec((CP, HID), lambda i: (0, 0)),
            pl.BlockSpec((1, CP), lambda i: (0, 0)),
        ],
        out_specs=pl.BlockSpec((C, BB), lambda i: (0, i)),
        out_shape=jax.ShapeDtypeStruct((C, B), jnp.float32),
    )(pooled_sum, W1, b1r, W2p, b2p)

    return out.T
